# Initial kernel scaffold; baseline (speedup 1.0000x reference)
import jax
import jax.numpy as jnp
from jax.experimental import pallas as pl


def kernel(x, table):
    raise NotImplementedError("write your pallas kernel here")


# speedup vs baseline: 1.0935x; 1.0935x over previous
import functools

import jax
import jax.numpy as jnp
from jax import lax
from jax.experimental import pallas as pl
from jax.experimental.pallas import tpu as pltpu
from jax.experimental.pallas import tpu_sc as plsc

D = 32
N = 16384 * 50
NC = 2
NS = 16
NW = NC * NS
PER_W = N // NW
G = 128
K = 8
C = G * K
CHUNKS = PER_W // C


def _gather_body(x_hbm, table_hbm, out_hbm, idx_v, rows_v, sem):
    wid = lax.axis_index("s") * NC + lax.axis_index("c")
    row0 = wid * (PER_W // G)

    def chunk(i, _):
        pltpu.sync_copy(x_hbm.at[pl.ds(row0 + i * K, K)], idx_v)
        copies = []
        for j in range(K):
            copies.append(
                pltpu.async_copy(
                    table_hbm.at[idx_v.at[j]],
                    rows_v.at[pl.ds(j * G, G)],
                    sem,
                )
            )
        for cp in copies:
            cp.wait()
        off = wid * PER_W + i * C
        pltpu.sync_copy(rows_v, out_hbm.at[pl.ds(off, C)])
        return _

    lax.fori_loop(0, CHUNKS, chunk, None)


@functools.partial(jax.jit, static_argnames=())
def kernel(x, table):
    B, L = x.shape
    xf = x.reshape(N // G, G).astype(jnp.int32)
    table = table.astype(jnp.float32)
    out = pl.kernel(
        _gather_body,
        out_type=jax.ShapeDtypeStruct((N, D), jnp.float32),
        mesh=plsc.VectorSubcoreMesh(core_axis_name="c", subcore_axis_name="s"),
        compiler_params=pltpu.CompilerParams(use_tc_tiling_on_sc=False),
        scratch_types=[
            pltpu.VMEM((K, G), jnp.int32),
            pltpu.VMEM((C, D), jnp.float32),
            pltpu.SemaphoreType.DMA,
        ],
    )(xf, table)
    return out.reshape(B, L, D)


# speedup vs baseline: 1.1122x; 1.0170x over previous
import functools

import jax
import jax.numpy as jnp
from jax import lax
from jax.experimental import pallas as pl
from jax.experimental.pallas import tpu as pltpu
from jax.experimental.pallas import tpu_sc as plsc

D = 32
N = 16384 * 50
NC = 2
NS = 16
NW = NC * NS
PER_W = N // NW
G = 128
K = 10
C = G * K
CHUNKS = PER_W // C
IDXROWS = PER_W // G


def _gather_body(x_hbm, table_hbm, out_hbm, idx_all, rows0, rows1,
                 gsem0, gsem1, wsem0, wsem1):
    wid = lax.axis_index("s") * NC + lax.axis_index("c")
    out_base = wid * PER_W
    rows = (rows0, rows1)
    gsem = (gsem0, gsem1)
    wsem = (wsem0, wsem1)

    pltpu.sync_copy(x_hbm.at[pl.ds(wid * IDXROWS, IDXROWS)], idx_all)

    def fire_gathers(c, b):
        for j in range(K):
            pltpu.async_copy(
                table_hbm.at[idx_all.at[c * K + j]],
                rows[b].at[pl.ds(j * G, G)],
                gsem[b],
            )

    def wait_gathers(b):
        pltpu.make_async_copy(
            table_hbm.at[pl.ds(0, C)], rows[b], gsem[b]
        ).wait()

    def write_async(c, b):
        pltpu.async_copy(
            rows[b], out_hbm.at[pl.ds(out_base + c * C, C)], wsem[b]
        )

    def wait_write(b):
        pltpu.make_async_copy(
            rows[b], out_hbm.at[pl.ds(0, C)], wsem[b]
        ).wait()

    def pair(i, _):
        for h in (0, 1):
            c = 2 * i + h

            @pl.when(i > 0)
            def _wait_buf():
                wait_write(h)

            fire_gathers(c, h)

            if h == 0:
                @pl.when(i > 0)
                def _drain_prev():
                    wait_gathers(1)
                    write_async(c - 1, 1)
            else:
                wait_gathers(0)
                write_async(c - 1, 0)
        return _

    lax.fori_loop(0, CHUNKS // 2, pair, None)

    wait_gathers(1)
    pltpu.sync_copy(rows[1], out_hbm.at[pl.ds(out_base + (CHUNKS - 1) * C, C)])
    wait_write(0)


@functools.partial(jax.jit, static_argnames=())
def kernel(x, table):
    B, L = x.shape
    xf = x.reshape(N // G, G).astype(jnp.int32)
    table = table.astype(jnp.float32)
    out = pl.kernel(
        _gather_body,
        out_type=jax.ShapeDtypeStruct((N, D), jnp.float32),
        mesh=plsc.VectorSubcoreMesh(core_axis_name="c", subcore_axis_name="s"),
        compiler_params=pltpu.CompilerParams(use_tc_tiling_on_sc=False),
        scratch_types=[
            pltpu.VMEM((IDXROWS, G), jnp.int32),
            pltpu.VMEM((C, D), jnp.float32),
            pltpu.VMEM((C, D), jnp.float32),
            pltpu.SemaphoreType.DMA,
            pltpu.SemaphoreType.DMA,
            pltpu.SemaphoreType.DMA,
            pltpu.SemaphoreType.DMA,
        ],
    )(xf, table)
    return out.reshape(B, L, D)


# speedup vs baseline: 1.7999x; 1.6184x over previous
import functools

import jax
import jax.numpy as jnp
from jax import lax
from jax.experimental import pallas as pl
from jax.experimental.pallas import tpu as pltpu
from jax.experimental.pallas import tpu_sc as plsc

B = 16384
L = 50
D = 32
NC = 2
NS = 16
NW = NC * NS
ROWS_W = B // NW
RC = 32
CHUNKS = ROWS_W // RC


def _gather_body(x_hbm, table_hbm, out_hbm, idx0, idx1, rows0, rows1,
                 gsem0, gsem1, wsem0, wsem1):
    wid = lax.axis_index("s") * NC + lax.axis_index("c")
    row_base = wid * ROWS_W
    idx = (idx0, idx1)
    rows = (rows0, rows1)
    gsem = (gsem0, gsem1)
    wsem = (wsem0, wsem1)

    def load_idx(c, b):
        pltpu.sync_copy(
            x_hbm.at[pl.ds(row_base + c * RC, RC)], idx[b]
        )

    def fire_gathers(b):
        for j in range(RC):
            pltpu.async_copy(
                table_hbm.at[idx[b].at[j]],
                rows[b].at[j],
                gsem[b],
            )

    def wait_gathers(b):
        pltpu.make_async_copy(
            out_hbm.at[pl.ds(0, RC)], rows[b], gsem[b]
        ).wait()

    def write_async(c, b):
        pltpu.async_copy(
            rows[b], out_hbm.at[pl.ds(row_base + c * RC, RC)], wsem[b]
        )

    def wait_write(b):
        pltpu.make_async_copy(
            rows[b], out_hbm.at[pl.ds(0, RC)], wsem[b]
        ).wait()

    def pair(i, _):
        for h in (0, 1):
            c = 2 * i + h

            @pl.when(i > 0)
            def _wait_buf():
                wait_write(h)

            load_idx(c, h)
            fire_gathers(h)

            if h == 0:
                @pl.when(i > 0)
                def _drain_prev():
                    wait_gathers(1)
                    write_async(c - 1, 1)
            else:
                wait_gathers(0)
                write_async(c - 1, 0)
        return _

    lax.fori_loop(0, CHUNKS // 2, pair, None)

    wait_gathers(1)
    pltpu.sync_copy(
        rows[1], out_hbm.at[pl.ds(row_base + (CHUNKS - 1) * RC, RC)]
    )
    wait_write(0)


@functools.partial(jax.jit, static_argnames=())
def kernel(x, table):
    out = pl.kernel(
        _gather_body,
        out_type=jax.ShapeDtypeStruct((B, L, D), jnp.float32),
        mesh=plsc.VectorSubcoreMesh(core_axis_name="c", subcore_axis_name="s"),
        compiler_params=pltpu.CompilerParams(use_tc_tiling_on_sc=False),
        scratch_types=[
            pltpu.VMEM((RC, L), jnp.int32),
            pltpu.VMEM((RC, L), jnp.int32),
            pltpu.VMEM((RC, L, D), jnp.float32),
            pltpu.VMEM((RC, L, D), jnp.float32),
            pltpu.SemaphoreType.DMA,
            pltpu.SemaphoreType.DMA,
            pltpu.SemaphoreType.DMA,
            pltpu.SemaphoreType.DMA,
        ],
    )(x.astype(jnp.int32), table.astype(jnp.float32))
    return out


# speedup vs baseline: 1.9375x; 1.0765x over previous
import functools

import jax
import jax.numpy as jnp
from jax import lax
from jax.experimental import pallas as pl
from jax.experimental.pallas import tpu as pltpu
from jax.experimental.pallas import tpu_sc as plsc

B = 16384
L = 50
D = 32
NC = 2
NS = 16
NW = NC * NS
COLS_W = B // NW
CC = 32
CHUNKS = COLS_W // CC
LG = 10


def _gather_body(xt_hbm, table_hbm, out_hbm, idx0, idx1, rows0, rows1,
                 gsem0, gsem1, wsem0, wsem1):
    wid = lax.axis_index("s") * NC + lax.axis_index("c")
    col_base = wid * COLS_W
    idx = (idx0, idx1)
    rows = (rows0, rows1)
    gsem = (gsem0, gsem1)
    wsem = (wsem0, wsem1)

    def load_idx(c, b):
        pltpu.sync_copy(
            xt_hbm.at[:, pl.ds(col_base + c * CC, CC)], idx[b]
        )

    def fire_gathers(b):
        def group(g, _):
            for j in range(LG):
                l = g * LG + j
                pltpu.async_copy(
                    table_hbm.at[idx[b].at[l]],
                    rows[b].at[l],
                    gsem[b],
                )
            return _
        lax.fori_loop(0, L // LG, group, None)

    def wait_gathers(b):
        pltpu.make_async_copy(
            out_hbm.at[:, pl.ds(0, CC)], rows[b], gsem[b]
        ).wait()

    def write_async(c, b):
        pltpu.async_copy(
            rows[b], out_hbm.at[:, pl.ds(col_base + c * CC, CC)], wsem[b]
        )

    def wait_write(b):
        pltpu.make_async_copy(
            rows[b], out_hbm.at[:, pl.ds(0, CC)], wsem[b]
        ).wait()

    def pair(i, _):
        for h in (0, 1):
            c = 2 * i + h

            @pl.when(i > 0)
            def _wait_buf():
                wait_write(h)

            load_idx(c, h)
            fire_gathers(h)

            if h == 0:
                @pl.when(i > 0)
                def _drain_prev():
                    wait_gathers(1)
                    write_async(c - 1, 1)
            else:
                wait_gathers(0)
                write_async(c - 1, 0)
        return _

    lax.fori_loop(0, CHUNKS // 2, pair, None)

    wait_gathers(1)
    pltpu.sync_copy(
        rows[1], out_hbm.at[:, pl.ds(col_base + (CHUNKS - 1) * CC, CC)]
    )
    wait_write(0)


@functools.partial(jax.jit, static_argnames=())
def kernel(x, table):
    xt = jnp.swapaxes(x, 0, 1).astype(jnp.int32)
    out = pl.kernel(
        _gather_body,
        out_type=jax.ShapeDtypeStruct((L, B, D), jnp.float32),
        mesh=plsc.VectorSubcoreMesh(core_axis_name="c", subcore_axis_name="s"),
        compiler_params=pltpu.CompilerParams(use_tc_tiling_on_sc=False),
        scratch_types=[
            pltpu.VMEM((L, CC), jnp.int32),
            pltpu.VMEM((L, CC), jnp.int32),
            pltpu.VMEM((L, CC, D), jnp.float32),
            pltpu.VMEM((L, CC, D), jnp.float32),
            pltpu.SemaphoreType.DMA,
            pltpu.SemaphoreType.DMA,
            pltpu.SemaphoreType.DMA,
            pltpu.SemaphoreType.DMA,
        ],
    )(xt, table.astype(jnp.float32))
    return jnp.swapaxes(out, 0, 1)
